# baseline (speedup 1.0000x reference)
import functools

import jax
import jax.numpy as jnp
import numpy as np
from jax import lax
from jax.experimental import pallas as pl
from jax.experimental.pallas import tpu as pltpu
from jax.experimental.pallas import tpu_sc as plsc

_CUTOFF = 5.0
_MAX_RADIAL = 8
_MAX_ANGULAR = 3


def _np_jn(l, x):
    x = np.asarray(x, dtype=np.float64)
    j0 = np.sin(x) / x
    if l == 0:
        return j0
    j1 = np.sin(x) / x**2 - np.cos(x) / x
    if l == 1:
        return j1
    jm, jc = j0, j1
    for ll in range(1, l):
        jm, jc = jc, (2 * ll + 1) / x * jc - jm
    return jc


def _np_bessel_zeros(l, z_max):
    xs = np.arange(0.5, z_max + 1.0, 0.005)
    vals = _np_jn(l, xs)
    idx = np.where(np.sign(vals[:-1]) * np.sign(vals[1:]) < 0)[0]
    zeros = []
    for k in idx:
        a, b = xs[k], xs[k + 1]
        fa = _np_jn(l, a)
        for _ in range(60):
            m = 0.5 * (a + b)
            fm = _np_jn(l, m)
            if fa * fm <= 0:
                b = m
            else:
                a, fa = m, fm
        z = 0.5 * (a + b)
        if z <= z_max:
            zeros.append(z)
    return np.asarray(zeros, dtype=np.float64)


_ZMAX = _MAX_RADIAL * np.pi + 1e-6
_ZEROS = [_np_bessel_zeros(l, _ZMAX) for l in range(_MAX_ANGULAR + 1)]
_N_PER_L = [int(z.shape[0]) for z in _ZEROS]
_M_PER_L = [2 * l + 1 for l in range(_MAX_ANGULAR + 1)]
_NORMS = [
    np.sqrt(2.0 / _CUTOFF**3) / np.abs(_np_jn(l + 1, _ZEROS[l]))
    for l in range(_MAX_ANGULAR + 1)
]
_NRAD = sum(_N_PER_L)
_NANG = sum(_M_PER_L)
_K = sum(m * n for m, n in zip(_M_PER_L, _N_PER_L))

_Z_ROW = np.concatenate(_ZEROS).astype(np.float32)[None, :]
_NRM_ROW = np.concatenate(_NORMS).astype(np.float32)[None, :]
_L_ROW = np.concatenate(
    [np.full(n, l, np.float32) for l, n in enumerate(_N_PER_L)]
)[None, :]

_KOFF_L = np.cumsum([0] + [m * n for m, n in zip(_M_PER_L, _N_PER_L)])
_ROFF_L = np.cumsum([0] + _N_PER_L)
_AOFF_L = np.cumsum([0] + _M_PER_L)

_REP_R = np.zeros((_NRAD, _K), np.float32)
_REP_A = np.zeros((_NANG, _K), np.float32)
for _l in range(_MAX_ANGULAR + 1):
    _m, _n = _M_PER_L[_l], _N_PER_L[_l]
    for _mm in range(_m):
        for _nn in range(_n):
            _k = _KOFF_L[_l] + _mm * _n + _nn
            _REP_R[_ROFF_L[_l] + _nn, _k] = 1.0
            _REP_A[_AOFF_L[_l] + _mm, _k] = 1.0

_KP = 128

_PK = 4
_RW = _NRAD * _PK
_OW = _KP * _PK

_Z28 = np.concatenate(_ZEROS).astype(np.float64)
_L28 = np.concatenate([np.full(n, l) for l, n in enumerate(_N_PER_L)])
_NRM28 = np.concatenate(_NORMS).astype(np.float64)

_S12 = np.zeros((12, _PK), np.float32)
for _i in range(_PK):
    _S12[3 * _i:3 * _i + 3, _i] = 1.0

_ZBLK = np.zeros((_PK, _RW), np.float32)
for _i in range(_PK):
    _ZBLK[_i, 28 * _i:28 * _i + 28] = (_Z28 / _CUTOFF).astype(np.float32)

_WLR = np.zeros((4, _RW), np.float32)
for _L in range(4):
    _row = np.where(_L28 == _L, _NRM28, 0.0).astype(np.float32)
    for _i in range(_PK):
        _WLR[_L, 28 * _i:28 * _i + 28] = _row

_CT = np.zeros((3, _PK, _RW), np.float32)
_DT = np.zeros((3, 1, _RW), np.float32)
for _t in range(3):
    _ge = (_L28 >= _t + 1).astype(np.float32)
    for _i in range(_PK):
        _CT[_t, _i, 28 * _i:28 * _i + 28] = _ge
    _DT[_t, 0, :] = np.tile(1.0 - _ge, _PK)

_REP_R4 = np.zeros((_RW, _OW), np.float32)
for _i in range(_PK):
    _REP_R4[28 * _i:28 * _i + 28, 128 * _i:128 * _i + _K] = _REP_R

def _poly_tables():
    def mono(e, c=1.0):
        return {e: c}

    def padd(a, b):
        out = dict(a)
        for k, v in b.items():
            out[k] = out.get(k, 0.0) + v
        return out

    def pmul(a, b):
        out = {}
        for ka, va in a.items():
            for kb, vb in b.items():
                k = tuple(x + y for x, y in zip(ka, kb))
                out[k] = out.get(k, 0.0) + va * vb
        return out

    def pscale(a, s):
        return {k: v * s for k, v in a.items()}

    X, Y, Z = mono((1, 0, 0)), mono((0, 1, 0)), mono((0, 0, 1))
    RR = padd(padd(pmul(X, X), pmul(Y, Y)), pmul(Z, Z))
    c1 = 0.4886025119029199
    c2a = 1.0925484305920792
    sh = [
        {(0, 0, 0): 0.28209479177387814},
        pscale(Y, c1), pscale(Z, c1), pscale(X, c1),
        pscale(pmul(X, Y), c2a),
        pscale(pmul(Y, Z), c2a),
        pscale(padd(pscale(pmul(Z, Z), 3.0), pscale(RR, -1.0)),
               0.31539156525252005),
        pscale(pmul(X, Z), c2a),
        pscale(padd(pmul(X, X), pscale(pmul(Y, Y), -1.0)), 0.5462742152960396),
        pscale(pmul(Y, padd(pscale(pmul(X, X), 3.0),
                            pscale(pmul(Y, Y), -1.0))), 0.5900435899266435),
        pscale(pmul(pmul(X, Y), Z), 2.890611442640554),
        pscale(pmul(Y, padd(pscale(pmul(Z, Z), 5.0), pscale(RR, -1.0))),
               0.4570457994644658),
        pscale(pmul(Z, padd(pscale(pmul(Z, Z), 5.0), pscale(RR, -3.0))),
               0.3731763325901154),
        pscale(pmul(X, padd(pscale(pmul(Z, Z), 5.0), pscale(RR, -1.0))),
               0.4570457994644658),
        pscale(pmul(Z, padd(pmul(X, X), pscale(pmul(Y, Y), -1.0))),
               1.445305721320277),
        pscale(pmul(X, padd(pmul(X, X), pscale(pmul(Y, Y), -3.0))),
               0.5900435899266435),
    ]
    return sh


_SH_POLY = _poly_tables()
_MONO3 = [(3, 0, 0), (2, 1, 0), (2, 0, 1), (1, 2, 0), (1, 1, 1),
          (1, 0, 2), (0, 3, 0), (0, 2, 1), (0, 1, 2), (0, 0, 3)]
_MONO2_IDX = {(2, 0, 0): 0, (1, 1, 0): 1, (1, 0, 1): 2,
              (0, 2, 0): 4, (0, 1, 1): 5, (0, 0, 2): 8}
_M3_SEL2 = [0, 0, 0, 4, 1, 8, 4, 4, 8, 8]
_M3_SEL1 = [0, 1, 2, 0, 2, 0, 1, 2, 1, 2]

_EA = np.zeros((12, 9 * _PK), np.float32)
_EB = np.zeros((12, 9 * _PK), np.float32)
for _i in range(_PK):
    for _a in range(3):
        for _b in range(3):
            _t = 3 * _a + _b
            _EA[3 * _i + _a, 9 * _i + _t] = 1.0
            _EB[3 * _i + _b, 9 * _i + _t] = 1.0

_FA = np.zeros((9 * _PK, 10 * _PK), np.float32)
_FB = np.zeros((12, 10 * _PK), np.float32)
for _i in range(_PK):
    for _t in range(10):
        _FA[9 * _i + _M3_SEL2[_t], 10 * _i + _t] = 1.0
        _FB[3 * _i + _M3_SEL1[_t], 10 * _i + _t] = 1.0

_ANG_OF_K = np.argmax(_REP_A, axis=0)
_C0ROW = np.zeros((1, _OW), np.float32)
_W1B = np.zeros((12, _OW), np.float32)
_W2B = np.zeros((9 * _PK, _OW), np.float32)
_W3B = np.zeros((10 * _PK, _OW), np.float32)
for _i in range(_PK):
    for _k in range(_K):
        _p = _SH_POLY[int(_ANG_OF_K[_k])]
        _col = 128 * _i + _k
        for _e, _cf in _p.items():
            _d = sum(_e)
            if _d == 0:
                _C0ROW[0, _col] += _cf
            elif _d == 1:
                _W1B[3 * _i + _e.index(1), _col] += _cf
            elif _d == 2:
                _W2B[9 * _i + _MONO2_IDX[_e], _col] += _cf
            else:
                _t3 = _MONO3.index(_e)
                _W3B[10 * _i + _t3, _col] += _cf

_G512 = np.zeros((4, _OW), np.float32)
for _c in range(4):
    _G512[_c, 128 * _c:128 * _c + 128] = 1.0

_P64 = np.float64(np.pi / 2)
_SC_P1 = np.float32(np.round(_P64 * 4096) / 4096)
_SC_P2 = np.float32(np.round((_P64 - np.float64(_SC_P1)) * 4096 ** 3)
                    / 4096 ** 3)
_SC_P3 = np.float32(_P64 - np.float64(_SC_P1) - np.float64(_SC_P2))
_TWO_OVER_PI = np.float32(2.0 / np.pi)

_EDGE_BLOCK = 1280
_N_EDGES = 160000
_N_NODES = 10000
_ACC_ROWS = 10240
_STRIPE = _ACC_ROWS // 16
_EPT = _N_EDGES // 16
_CHUNK = 128
_NFULL = _EPT // _CHUNK
_TAIL = _EPT - _NFULL * _CHUNK


def _dot(a, b):
    return jnp.dot(a, b, preferred_element_type=jnp.float32,
                   precision=jax.lax.Precision.HIGHEST)


def _sincos(xr):
    k = jnp.round(xr * _TWO_OVER_PI)
    ki = k.astype(jnp.int32)
    y = ((xr - k * _SC_P1) - k * _SC_P2) - k * _SC_P3
    y2 = y * y
    sp = y * (1.0 + y2 * (np.float32(-1 / 6) + y2 * (np.float32(1 / 120)
              + y2 * (np.float32(-1 / 5040) + y2 * np.float32(1 / 362880)))))
    cp = 1.0 + y2 * (np.float32(-0.5) + y2 * (np.float32(1 / 24)
              + y2 * (np.float32(-1 / 720) + y2 * np.float32(1 / 40320))))
    swap = (ki & 1) == 1
    s_sgn = jnp.where((ki & 2) == 2, -1.0, 1.0)
    c_sgn = jnp.where(((ki + 1) & 2) == 2, -1.0, 1.0)
    return s_sgn * jnp.where(swap, cp, sp), c_sgn * jnp.where(swap, sp, cp)


def _edge_kernel(r12_ref, spec_ref, s12_ref, zblk_ref, wlr_ref, ct0_ref,
                 ct1_ref, ct2_ref, dt_ref, repr4_ref, ea_ref, eb_ref, fa_ref, fb_ref, c0_ref,
                 w1b_ref, w2b_ref, w3b_ref, g512_ref,
                 e0_ref, e1_ref, e2_ref, e3_ref):
    B = e0_ref.shape[0]
    R12 = r12_ref[...]
    r2 = _dot(R12 * R12, s12_ref[...])
    r = jnp.sqrt(r2)
    rinv = 1.0 / jnp.maximum(r, 1e-9)

    xr = jnp.maximum(_dot(r, zblk_ref[...]), 1e-2)
    s, c = _sincos(xr)
    inv = 1.0 / xr
    j0 = s * inv
    j1 = s * inv * inv - c * inv
    j2 = 3.0 * inv * j1 - j0
    j3 = 5.0 * inv * j2 - j1
    wlr = wlr_ref[...]
    rad = (j0 * wlr[0:1, :] + j1 * wlr[1:2, :]
           + j2 * wlr[2:3, :] + j3 * wlr[3:4, :])
    dt = dt_ref[...]
    f = ((_dot(rinv, ct0_ref[...]) + dt[0:1, :])
         * (_dot(rinv, ct1_ref[...]) + dt[1:2, :])
         * (_dot(rinv, ct2_ref[...]) + dt[2:3, :]))
    rad_ext = _dot(rad * f, repr4_ref[...])

    m2 = _dot(R12, ea_ref[...]) * _dot(R12, eb_ref[...])
    m3 = _dot(m2, fa_ref[...]) * _dot(R12, fb_ref[...])
    ang = (c0_ref[...] + _dot(R12, w1b_ref[...])
           + _dot(m2, w2b_ref[...]) + _dot(m3, w3b_ref[...]))

    u = jnp.reshape(rad_ext * ang, (B, _KP))
    spec4 = spec_ref[...][:, 0:4]
    sb = _dot(spec4, g512_ref[...])
    e0_ref[...] = u * sb[:, 0:128]
    e1_ref[...] = u * sb[:, 128:256]
    e2_ref[...] = u * sb[:, 256:384]
    e3_ref[...] = u * sb[:, 384:512]


def _const_spec(shape):
    return pl.BlockSpec(shape, lambda b: (0,) * len(shape))


def _edge_features(R_ij, spec_e):
    P = R_ij.shape[0]
    B = _EDGE_BLOCK
    B4 = B // _PK
    grid = (P // B,)
    R12 = R_ij.reshape(P // _PK, 3 * _PK)
    out_spec = pl.BlockSpec((B, _KP), lambda b: (b, 0))
    e_shape = jax.ShapeDtypeStruct((P, _KP), jnp.float32)

    return pl.pallas_call(
        _edge_kernel,
        grid=grid,
        in_specs=[
            pl.BlockSpec((B4, 12), lambda b: (b, 0)),
            pl.BlockSpec((B, 128), lambda b: (b, 0)),
            _const_spec((12, _PK)),
            _const_spec((_PK, _RW)),
            _const_spec((4, _RW)),
            _const_spec((_PK, _RW)),
            _const_spec((_PK, _RW)),
            _const_spec((_PK, _RW)),
            _const_spec((3, _RW)),
            _const_spec((_RW, _OW)),
            _const_spec((12, 9 * _PK)),
            _const_spec((12, 9 * _PK)),
            _const_spec((9 * _PK, 10 * _PK)),
            _const_spec((12, 10 * _PK)),
            _const_spec((1, _OW)),
            _const_spec((12, _OW)),
            _const_spec((9 * _PK, _OW)),
            _const_spec((10 * _PK, _OW)),
            _const_spec((4, _OW)),
        ],
        out_specs=[out_spec] * 4,
        out_shape=[e_shape] * 4,
    )(
        R12,
        spec_e,
        jnp.asarray(_S12),
        jnp.asarray(_ZBLK),
        jnp.asarray(_WLR),
        jnp.asarray(_CT[0]),
        jnp.asarray(_CT[1]),
        jnp.asarray(_CT[2]),
        jnp.asarray(_DT[:, 0, :]),
        jnp.asarray(_REP_R4),
        jnp.asarray(_EA),
        jnp.asarray(_EB),
        jnp.asarray(_FA),
        jnp.asarray(_FB),
        jnp.asarray(_C0ROW),
        jnp.asarray(_W1B),
        jnp.asarray(_W2B),
        jnp.asarray(_W3B),
        jnp.asarray(_G512),
    )


_GW = 32
_GEPT = _N_EDGES // _GW
_GCHUNK = 128
_GNFULL = _GEPT // _GCHUNK
_GTAIL = _GEPT - _GNFULL * _GCHUNK


def _sc_gather_body(tbl_hbm, idx_hbm, out_hbm, idx_v, rows_v, idx_t, rows_t,
                    sem):
    cid = lax.axis_index("c")
    sid = lax.axis_index("s")
    wid = sid * 2 + cid
    base = wid * _GEPT

    def body(g, _):
        off = base + g * _GCHUNK
        pltpu.sync_copy(idx_hbm.at[pl.ds(off, _GCHUNK)], idx_v)
        pltpu.async_copy(tbl_hbm.at[idx_v], rows_v, sem).wait()
        pltpu.sync_copy(rows_v, out_hbm.at[pl.ds(off, _GCHUNK)])
        return _

    lax.fori_loop(0, _GNFULL, body, 0)
    toff = base + _GNFULL * _GCHUNK
    pltpu.sync_copy(idx_hbm.at[pl.ds(toff, _GTAIL)], idx_t)
    pltpu.async_copy(tbl_hbm.at[idx_t], rows_t, sem).wait()
    pltpu.sync_copy(rows_t, out_hbm.at[pl.ds(toff, _GTAIL)])


def _sc_gather(T16, j):
    mesh = plsc.VectorSubcoreMesh(core_axis_name="c", subcore_axis_name="s")
    f = pl.kernel(
        _sc_gather_body,
        mesh=mesh,
        out_type=jax.ShapeDtypeStruct((_N_EDGES, 128), jnp.float32),
        scratch_types=[
            pltpu.VMEM((_GCHUNK,), jnp.int32),
            pltpu.VMEM((_GCHUNK, 128), jnp.float32),
            pltpu.VMEM((_GTAIL,), jnp.int32),
            pltpu.VMEM((_GTAIL, 128), jnp.float32),
            pltpu.SemaphoreType.DMA,
        ],
    )
    return f(T16, j)


def _sc_scatter_body(e0, e1, e2, e3, idx_hbm, zeros_hbm,
                     out0, out1, out2, out3,
                     idx_v, rows_v, idx_t, rows_t, acc_sh):
    cid = lax.axis_index("c")
    sid = lax.axis_index("s")
    e_refs = (e0, e1, e2, e3)
    out_refs = (out0, out1, out2, out3)
    base_t = sid * _EPT

    for pass_k in range(2):
        pltpu.sync_copy(zeros_hbm, acc_sh.at[pl.ds(sid * _STRIPE, _STRIPE)])
        plsc.subcore_barrier()

        for which in range(2):
            ch = 2 * pass_k + which
            e_ref = e_refs[ch]

            @pl.when(cid == which)
            def _scatter(e_ref=e_ref):
                def body(c, _):
                    off = base_t + c * _CHUNK
                    pltpu.sync_copy(idx_hbm.at[pl.ds(off, _CHUNK)], idx_v)
                    pltpu.sync_copy(e_ref.at[pl.ds(off, _CHUNK)], rows_v)
                    pltpu.sync_copy(rows_v, acc_sh.at[idx_v], add=True)
                    return _

                lax.fori_loop(0, _NFULL, body, 0)
                toff = base_t + _NFULL * _CHUNK
                pltpu.sync_copy(idx_hbm.at[pl.ds(toff, _TAIL)], idx_t)
                pltpu.sync_copy(e_ref.at[pl.ds(toff, _TAIL)], rows_t)
                pltpu.sync_copy(rows_t, acc_sh.at[idx_t], add=True)

        plsc.subcore_barrier()

        for which in range(2):
            ch = 2 * pass_k + which
            out_ref = out_refs[ch]

            @pl.when(cid == which)
            def _flush(out_ref=out_ref):
                s = pl.ds(sid * _STRIPE, _STRIPE)
                pltpu.sync_copy(acc_sh.at[s], out_ref.at[s])

        plsc.subcore_barrier()


def _sc_scatter(e0, e1, e2, e3, i):
    zeros = jnp.zeros((_STRIPE, _KP), jnp.float32)
    acc_ty = jax.ShapeDtypeStruct((_ACC_ROWS, _KP), jnp.float32)
    mesh = plsc.VectorSubcoreMesh(core_axis_name="c", subcore_axis_name="s")
    f = pl.kernel(
        _sc_scatter_body,
        mesh=mesh,
        out_type=[acc_ty] * 4,
        scratch_types=[
            pltpu.VMEM((_CHUNK,), jnp.int32),
            pltpu.VMEM((_CHUNK, _KP), jnp.float32),
            pltpu.VMEM((_TAIL,), jnp.int32),
            pltpu.VMEM((_TAIL, _KP), jnp.float32),
            pltpu.VMEM_SHARED((_ACC_ROWS, _KP), jnp.float32),
        ],
    )
    return f(e0, e1, e2, e3, i, zeros)


@functools.partial(jax.jit)
def kernel(R_ij, i, j, species, species_embedding):
    onehot = (species[:, None] == jnp.arange(8, dtype=species.dtype)[None, :])
    T = jnp.dot(onehot.astype(jnp.float32), species_embedding)
    T128 = jnp.concatenate([T, jnp.zeros((T.shape[0], 124), jnp.float32)], axis=1)
    spec128 = _sc_gather(T128, j)
    e0, e1, e2, e3 = _edge_features(R_ij, spec128)
    acc = _sc_scatter(e0, e1, e2, e3, i)

    N = species.shape[0]
    leaves = []
    for l in range(_MAX_ANGULAR + 1):
        m, n = _M_PER_L[l], _N_PER_L[l]
        off = int(_KOFF_L[l])
        sl = [a[:N, off:off + m * n] for a in acc]
        leaf = jnp.stack(sl, axis=-1).reshape(N, m, n, 4)
        leaves.append(leaf)
    return tuple(leaves)


# speedup vs baseline: 1.4182x; 1.4182x over previous
import functools

import jax
import jax.numpy as jnp
import numpy as np
from jax import lax
from jax.experimental import pallas as pl
from jax.experimental.pallas import tpu as pltpu
from jax.experimental.pallas import tpu_sc as plsc

_CUTOFF = 5.0
_MAX_RADIAL = 8
_MAX_ANGULAR = 3


def _np_jn(l, x):
    x = np.asarray(x, dtype=np.float64)
    j0 = np.sin(x) / x
    if l == 0:
        return j0
    j1 = np.sin(x) / x**2 - np.cos(x) / x
    if l == 1:
        return j1
    jm, jc = j0, j1
    for ll in range(1, l):
        jm, jc = jc, (2 * ll + 1) / x * jc - jm
    return jc


def _np_bessel_zeros(l, z_max):
    xs = np.arange(0.5, z_max + 1.0, 0.005)
    vals = _np_jn(l, xs)
    idx = np.where(np.sign(vals[:-1]) * np.sign(vals[1:]) < 0)[0]
    zeros = []
    for k in idx:
        a, b = xs[k], xs[k + 1]
        fa = _np_jn(l, a)
        for _ in range(60):
            m = 0.5 * (a + b)
            fm = _np_jn(l, m)
            if fa * fm <= 0:
                b = m
            else:
                a, fa = m, fm
        z = 0.5 * (a + b)
        if z <= z_max:
            zeros.append(z)
    return np.asarray(zeros, dtype=np.float64)


_ZMAX = _MAX_RADIAL * np.pi + 1e-6
_ZEROS = [_np_bessel_zeros(l, _ZMAX) for l in range(_MAX_ANGULAR + 1)]
_N_PER_L = [int(z.shape[0]) for z in _ZEROS]
_M_PER_L = [2 * l + 1 for l in range(_MAX_ANGULAR + 1)]
_NORMS = [
    np.sqrt(2.0 / _CUTOFF**3) / np.abs(_np_jn(l + 1, _ZEROS[l]))
    for l in range(_MAX_ANGULAR + 1)
]
_NRAD = sum(_N_PER_L)
_NANG = sum(_M_PER_L)
_K = sum(m * n for m, n in zip(_M_PER_L, _N_PER_L))

_Z_ROW = np.concatenate(_ZEROS).astype(np.float32)[None, :]
_NRM_ROW = np.concatenate(_NORMS).astype(np.float32)[None, :]
_L_ROW = np.concatenate(
    [np.full(n, l, np.float32) for l, n in enumerate(_N_PER_L)]
)[None, :]

_KOFF_L = np.cumsum([0] + [m * n for m, n in zip(_M_PER_L, _N_PER_L)])
_ROFF_L = np.cumsum([0] + _N_PER_L)
_AOFF_L = np.cumsum([0] + _M_PER_L)

_REP_R = np.zeros((_NRAD, _K), np.float32)
_REP_A = np.zeros((_NANG, _K), np.float32)
for _l in range(_MAX_ANGULAR + 1):
    _m, _n = _M_PER_L[_l], _N_PER_L[_l]
    for _mm in range(_m):
        for _nn in range(_n):
            _k = _KOFF_L[_l] + _mm * _n + _nn
            _REP_R[_ROFF_L[_l] + _nn, _k] = 1.0
            _REP_A[_AOFF_L[_l] + _mm, _k] = 1.0

_KP = 128

_PK = 4
_RW = _NRAD * _PK
_OW = _KP * _PK

_Z28 = np.concatenate(_ZEROS).astype(np.float64)
_L28 = np.concatenate([np.full(n, l) for l, n in enumerate(_N_PER_L)])
_NRM28 = np.concatenate(_NORMS).astype(np.float64)

_S12 = np.zeros((12, _PK), np.float32)
for _i in range(_PK):
    _S12[3 * _i:3 * _i + 3, _i] = 1.0

_CB = np.zeros((_PK, _RW), np.float32)
for _i in range(_PK):
    _CB[_i, 28 * _i:28 * _i + 28] = 1.0
_ZROW112 = np.tile((_Z28 / _CUTOFF).astype(np.float32), _PK)[None, :]

_WLR = np.zeros((4, _RW), np.float32)
for _L in range(4):
    _row = np.where(_L28 == _L, _NRM28, 0.0).astype(np.float32)
    for _i in range(_PK):
        _WLR[_L, 28 * _i:28 * _i + 28] = _row

_CT3 = np.zeros((3, _RW), np.float32)
_DT3 = np.zeros((3, _RW), np.float32)
for _t in range(3):
    _ge = (_L28 >= _t + 1).astype(np.float32)
    _CT3[_t, :] = np.tile(_ge, _PK)
    _DT3[_t, :] = np.tile(1.0 - _ge, _PK)

_REP_R4 = np.zeros((_RW, _OW), np.float32)
for _i in range(_PK):
    _REP_R4[28 * _i:28 * _i + 28, 128 * _i:128 * _i + _K] = _REP_R

def _poly_tables():
    def mono(e, c=1.0):
        return {e: c}

    def padd(a, b):
        out = dict(a)
        for k, v in b.items():
            out[k] = out.get(k, 0.0) + v
        return out

    def pmul(a, b):
        out = {}
        for ka, va in a.items():
            for kb, vb in b.items():
                k = tuple(x + y for x, y in zip(ka, kb))
                out[k] = out.get(k, 0.0) + va * vb
        return out

    def pscale(a, s):
        return {k: v * s for k, v in a.items()}

    X, Y, Z = mono((1, 0, 0)), mono((0, 1, 0)), mono((0, 0, 1))
    RR = padd(padd(pmul(X, X), pmul(Y, Y)), pmul(Z, Z))
    c1 = 0.4886025119029199
    c2a = 1.0925484305920792
    sh = [
        {(0, 0, 0): 0.28209479177387814},
        pscale(Y, c1), pscale(Z, c1), pscale(X, c1),
        pscale(pmul(X, Y), c2a),
        pscale(pmul(Y, Z), c2a),
        pscale(padd(pscale(pmul(Z, Z), 3.0), pscale(RR, -1.0)),
               0.31539156525252005),
        pscale(pmul(X, Z), c2a),
        pscale(padd(pmul(X, X), pscale(pmul(Y, Y), -1.0)), 0.5462742152960396),
        pscale(pmul(Y, padd(pscale(pmul(X, X), 3.0),
                            pscale(pmul(Y, Y), -1.0))), 0.5900435899266435),
        pscale(pmul(pmul(X, Y), Z), 2.890611442640554),
        pscale(pmul(Y, padd(pscale(pmul(Z, Z), 5.0), pscale(RR, -1.0))),
               0.4570457994644658),
        pscale(pmul(Z, padd(pscale(pmul(Z, Z), 5.0), pscale(RR, -3.0))),
               0.3731763325901154),
        pscale(pmul(X, padd(pscale(pmul(Z, Z), 5.0), pscale(RR, -1.0))),
               0.4570457994644658),
        pscale(pmul(Z, padd(pmul(X, X), pscale(pmul(Y, Y), -1.0))),
               1.445305721320277),
        pscale(pmul(X, padd(pmul(X, X), pscale(pmul(Y, Y), -3.0))),
               0.5900435899266435),
    ]
    return sh


_SH_POLY = _poly_tables()
_MONO3 = [(3, 0, 0), (2, 1, 0), (2, 0, 1), (1, 2, 0), (1, 1, 1),
          (1, 0, 2), (0, 3, 0), (0, 2, 1), (0, 1, 2), (0, 0, 3)]
_MONO2_IDX = {(2, 0, 0): 0, (1, 1, 0): 1, (1, 0, 1): 2,
              (0, 2, 0): 4, (0, 1, 1): 5, (0, 0, 2): 8}
_M3_SEL2 = [0, 0, 0, 4, 1, 8, 4, 4, 8, 8]
_M3_SEL1 = [0, 1, 2, 0, 2, 0, 1, 2, 1, 2]

_EA = np.zeros((12, 9 * _PK), np.float32)
_EB = np.zeros((12, 9 * _PK), np.float32)
for _i in range(_PK):
    for _a in range(3):
        for _b in range(3):
            _t = 3 * _a + _b
            _EA[3 * _i + _a, 9 * _i + _t] = 1.0
            _EB[3 * _i + _b, 9 * _i + _t] = 1.0

_FA = np.zeros((9 * _PK, 10 * _PK), np.float32)
_FB = np.zeros((12, 10 * _PK), np.float32)
for _i in range(_PK):
    for _t in range(10):
        _FA[9 * _i + _M3_SEL2[_t], 10 * _i + _t] = 1.0
        _FB[3 * _i + _M3_SEL1[_t], 10 * _i + _t] = 1.0

_ANG_OF_K = np.argmax(_REP_A, axis=0)
_C0ROW = np.zeros((1, _OW), np.float32)
_W1B = np.zeros((12, _OW), np.float32)
_W2B = np.zeros((9 * _PK, _OW), np.float32)
_W3B = np.zeros((10 * _PK, _OW), np.float32)
for _i in range(_PK):
    for _k in range(_K):
        _p = _SH_POLY[int(_ANG_OF_K[_k])]
        _col = 128 * _i + _k
        for _e, _cf in _p.items():
            _d = sum(_e)
            if _d == 0:
                _C0ROW[0, _col] += _cf
            elif _d == 1:
                _W1B[3 * _i + _e.index(1), _col] += _cf
            elif _d == 2:
                _W2B[9 * _i + _MONO2_IDX[_e], _col] += _cf
            else:
                _t3 = _MONO3.index(_e)
                _W3B[10 * _i + _t3, _col] += _cf

_G512 = np.zeros((4, _OW), np.float32)
for _c in range(4):
    _G512[_c, 128 * _c:128 * _c + 128] = 1.0


def _bf_split_np(m):
    hi = m.astype(np.float32).astype(jnp.bfloat16).astype(np.float32)
    return hi, (m - hi).astype(np.float32)


_W1B_H, _W1B_L = _bf_split_np(_W1B)
_W2B_H, _W2B_L = _bf_split_np(_W2B)
_W3B_H, _W3B_L = _bf_split_np(_W3B)

_P64 = np.float64(np.pi / 2)
_SC_P1 = np.float32(np.round(_P64 * 4096) / 4096)
_SC_P2 = np.float32(np.round((_P64 - np.float64(_SC_P1)) * 4096 ** 3)
                    / 4096 ** 3)
_SC_P3 = np.float32(_P64 - np.float64(_SC_P1) - np.float64(_SC_P2))
_TWO_OVER_PI = np.float32(2.0 / np.pi)

_EDGE_BLOCK = 1280
_N_EDGES = 160000
_N_NODES = 10000
_ACC_ROWS = 10240
_STRIPE = _ACC_ROWS // 16
_EPT = _N_EDGES // 16
_CHUNK = 128
_NFULL = _EPT // _CHUNK
_TAIL = _EPT - _NFULL * _CHUNK


def _dot(a, b):
    return jnp.dot(a, b, preferred_element_type=jnp.float32)


def _bsplit(a):
    hi = a.astype(jnp.bfloat16).astype(jnp.float32)
    return hi, a - hi


def _pick(ah, al, m):
    return _dot(ah, m) + _dot(al, m)


def _coef(ah, al, mh, ml):
    return _dot(ah, mh) + _dot(al, mh) + _dot(ah, ml)


def _sincos(xr):
    k = jnp.round(xr * _TWO_OVER_PI)
    ki = k.astype(jnp.int32)
    y = ((xr - k * _SC_P1) - k * _SC_P2) - k * _SC_P3
    y2 = y * y
    sp = y * (1.0 + y2 * (np.float32(-1 / 6) + y2 * (np.float32(1 / 120)
              + y2 * (np.float32(-1 / 5040) + y2 * np.float32(1 / 362880)))))
    cp = 1.0 + y2 * (np.float32(-0.5) + y2 * (np.float32(1 / 24)
              + y2 * (np.float32(-1 / 720) + y2 * np.float32(1 / 40320))))
    swap = (ki & 1) == 1
    s_sgn = jnp.where((ki & 2) == 2, -1.0, 1.0)
    c_sgn = jnp.where(((ki + 1) & 2) == 2, -1.0, 1.0)
    return s_sgn * jnp.where(swap, cp, sp), c_sgn * jnp.where(swap, sp, cp)


def _edge_kernel(r12_ref, spec_ref, s12_ref, cb_ref, zrow_ref, wlr_ref,
                 ct3_ref, dt3_ref, repr4_ref, ea_ref, eb_ref, fa_ref, fb_ref,
                 c0_ref, w1bh_ref, w1bl_ref, w2bh_ref, w2bl_ref, w3bh_ref,
                 w3bl_ref, g512_ref, e0_ref, e1_ref, e2_ref, e3_ref):
    B = e0_ref.shape[0]
    R12 = r12_ref[...]
    Rh, Rl = _bsplit(R12)
    sqh, sql = _bsplit(R12 * R12)
    r2 = _pick(sqh, sql, s12_ref[...])
    r = jnp.sqrt(r2)
    rinv = 1.0 / jnp.maximum(r, 1e-9)
    cb = cb_ref[...]

    rh, rl = _bsplit(r)
    r_ext = _pick(rh, rl, cb)
    xr = jnp.maximum(r_ext * zrow_ref[...], 1e-2)
    s, c = _sincos(xr)
    inv = 1.0 / xr
    j0 = s * inv
    j1 = s * inv * inv - c * inv
    j2 = 3.0 * inv * j1 - j0
    j3 = 5.0 * inv * j2 - j1
    wlr = wlr_ref[...]
    rad = (j0 * wlr[0:1, :] + j1 * wlr[1:2, :]
           + j2 * wlr[2:3, :] + j3 * wlr[3:4, :])
    rih, ril = _bsplit(rinv)
    ri_ext = _pick(rih, ril, cb)
    ct3 = ct3_ref[...]
    dt3 = dt3_ref[...]
    f = ((ri_ext * ct3[0:1, :] + dt3[0:1, :])
         * (ri_ext * ct3[1:2, :] + dt3[1:2, :])
         * (ri_ext * ct3[2:3, :] + dt3[2:3, :]))
    radh, radl_ = _bsplit(rad * f)
    rad_ext = _pick(radh, radl_, repr4_ref[...])

    m2 = _pick(Rh, Rl, ea_ref[...]) * _pick(Rh, Rl, eb_ref[...])
    m2h, m2l = _bsplit(m2)
    m3 = _pick(m2h, m2l, fa_ref[...]) * _pick(Rh, Rl, fb_ref[...])
    m3h, m3l = _bsplit(m3)
    ang = (c0_ref[...]
           + _coef(Rh, Rl, w1bh_ref[...], w1bl_ref[...])
           + _coef(m2h, m2l, w2bh_ref[...], w2bl_ref[...])
           + _coef(m3h, m3l, w3bh_ref[...], w3bl_ref[...]))

    u = jnp.reshape(rad_ext * ang, (B, _KP))
    spec4 = spec_ref[...][:, 0:4]
    sph, spl = _bsplit(spec4)
    sb = _pick(sph, spl, g512_ref[...])
    e0_ref[...] = u * sb[:, 0:128]
    e1_ref[...] = u * sb[:, 128:256]
    e2_ref[...] = u * sb[:, 256:384]
    e3_ref[...] = u * sb[:, 384:512]


def _const_spec(shape):
    return pl.BlockSpec(shape, lambda b: (0,) * len(shape))


def _edge_features(R_ij, spec_e):
    P = R_ij.shape[0]
    B = _EDGE_BLOCK
    B4 = B // _PK
    grid = (P // B,)
    R12 = R_ij.reshape(P // _PK, 3 * _PK)
    out_spec = pl.BlockSpec((B, _KP), lambda b: (b, 0))
    e_shape = jax.ShapeDtypeStruct((P, _KP), jnp.float32)

    return pl.pallas_call(
        _edge_kernel,
        grid=grid,
        in_specs=[
            pl.BlockSpec((B4, 12), lambda b: (b, 0)),
            pl.BlockSpec((B, 128), lambda b: (b, 0)),
            _const_spec((12, _PK)),
            _const_spec((_PK, _RW)),
            _const_spec((1, _RW)),
            _const_spec((4, _RW)),
            _const_spec((3, _RW)),
            _const_spec((3, _RW)),
            _const_spec((_RW, _OW)),
            _const_spec((12, 9 * _PK)),
            _const_spec((12, 9 * _PK)),
            _const_spec((9 * _PK, 10 * _PK)),
            _const_spec((12, 10 * _PK)),
            _const_spec((1, _OW)),
            _const_spec((12, _OW)),
            _const_spec((12, _OW)),
            _const_spec((9 * _PK, _OW)),
            _const_spec((9 * _PK, _OW)),
            _const_spec((10 * _PK, _OW)),
            _const_spec((10 * _PK, _OW)),
            _const_spec((4, _OW)),
        ],
        out_specs=[out_spec] * 4,
        out_shape=[e_shape] * 4,
    )(
        R12,
        spec_e,
        jnp.asarray(_S12),
        jnp.asarray(_CB),
        jnp.asarray(_ZROW112),
        jnp.asarray(_WLR),
        jnp.asarray(_CT3),
        jnp.asarray(_DT3),
        jnp.asarray(_REP_R4),
        jnp.asarray(_EA),
        jnp.asarray(_EB),
        jnp.asarray(_FA),
        jnp.asarray(_FB),
        jnp.asarray(_C0ROW),
        jnp.asarray(_W1B_H),
        jnp.asarray(_W1B_L),
        jnp.asarray(_W2B_H),
        jnp.asarray(_W2B_L),
        jnp.asarray(_W3B_H),
        jnp.asarray(_W3B_L),
        jnp.asarray(_G512),
    )


_GW = 32
_GEPT = _N_EDGES // _GW
_GCHUNK = 128
_GNFULL = _GEPT // _GCHUNK
_GTAIL = _GEPT - _GNFULL * _GCHUNK


def _sc_gather_body(tbl_hbm, idx_hbm, out_hbm, idx_v, rows_v, idx_t, rows_t,
                    sem):
    cid = lax.axis_index("c")
    sid = lax.axis_index("s")
    wid = sid * 2 + cid
    base = wid * _GEPT

    def body(g, _):
        off = base + g * _GCHUNK
        pltpu.sync_copy(idx_hbm.at[pl.ds(off, _GCHUNK)], idx_v)
        pltpu.async_copy(tbl_hbm.at[idx_v], rows_v, sem).wait()
        pltpu.sync_copy(rows_v, out_hbm.at[pl.ds(off, _GCHUNK)])
        return _

    lax.fori_loop(0, _GNFULL, body, 0)
    toff = base + _GNFULL * _GCHUNK
    pltpu.sync_copy(idx_hbm.at[pl.ds(toff, _GTAIL)], idx_t)
    pltpu.async_copy(tbl_hbm.at[idx_t], rows_t, sem).wait()
    pltpu.sync_copy(rows_t, out_hbm.at[pl.ds(toff, _GTAIL)])


def _sc_gather(T16, j):
    mesh = plsc.VectorSubcoreMesh(core_axis_name="c", subcore_axis_name="s")
    f = pl.kernel(
        _sc_gather_body,
        mesh=mesh,
        out_type=jax.ShapeDtypeStruct((_N_EDGES, 128), jnp.float32),
        scratch_types=[
            pltpu.VMEM((_GCHUNK,), jnp.int32),
            pltpu.VMEM((_GCHUNK, 128), jnp.float32),
            pltpu.VMEM((_GTAIL,), jnp.int32),
            pltpu.VMEM((_GTAIL, 128), jnp.float32),
            pltpu.SemaphoreType.DMA,
        ],
    )
    return f(T16, j)


def _sc_scatter_body(e0, e1, e2, e3, idx_hbm, zeros_hbm,
                     out0, out1, out2, out3,
                     idx_v, rows_v, idx_t, rows_t, acc_sh):
    cid = lax.axis_index("c")
    sid = lax.axis_index("s")
    e_refs = (e0, e1, e2, e3)
    out_refs = (out0, out1, out2, out3)
    base_t = sid * _EPT

    for pass_k in range(2):
        pltpu.sync_copy(zeros_hbm, acc_sh.at[pl.ds(sid * _STRIPE, _STRIPE)])
        plsc.subcore_barrier()

        for which in range(2):
            ch = 2 * pass_k + which
            e_ref = e_refs[ch]

            @pl.when(cid == which)
            def _scatter(e_ref=e_ref):
                def body(c, _):
                    off = base_t + c * _CHUNK
                    pltpu.sync_copy(idx_hbm.at[pl.ds(off, _CHUNK)], idx_v)
                    pltpu.sync_copy(e_ref.at[pl.ds(off, _CHUNK)], rows_v)
                    pltpu.sync_copy(rows_v, acc_sh.at[idx_v], add=True)
                    return _

                lax.fori_loop(0, _NFULL, body, 0)
                toff = base_t + _NFULL * _CHUNK
                pltpu.sync_copy(idx_hbm.at[pl.ds(toff, _TAIL)], idx_t)
                pltpu.sync_copy(e_ref.at[pl.ds(toff, _TAIL)], rows_t)
                pltpu.sync_copy(rows_t, acc_sh.at[idx_t], add=True)

        plsc.subcore_barrier()

        for which in range(2):
            ch = 2 * pass_k + which
            out_ref = out_refs[ch]

            @pl.when(cid == which)
            def _flush(out_ref=out_ref):
                s = pl.ds(sid * _STRIPE, _STRIPE)
                pltpu.sync_copy(acc_sh.at[s], out_ref.at[s])

        plsc.subcore_barrier()


def _sc_scatter(e0, e1, e2, e3, i):
    zeros = jnp.zeros((_STRIPE, _KP), jnp.float32)
    acc_ty = jax.ShapeDtypeStruct((_ACC_ROWS, _KP), jnp.float32)
    mesh = plsc.VectorSubcoreMesh(core_axis_name="c", subcore_axis_name="s")
    f = pl.kernel(
        _sc_scatter_body,
        mesh=mesh,
        out_type=[acc_ty] * 4,
        scratch_types=[
            pltpu.VMEM((_CHUNK,), jnp.int32),
            pltpu.VMEM((_CHUNK, _KP), jnp.float32),
            pltpu.VMEM((_TAIL,), jnp.int32),
            pltpu.VMEM((_TAIL, _KP), jnp.float32),
            pltpu.VMEM_SHARED((_ACC_ROWS, _KP), jnp.float32),
        ],
    )
    return f(e0, e1, e2, e3, i, zeros)


@functools.partial(jax.jit)
def kernel(R_ij, i, j, species, species_embedding):
    onehot = (species[:, None] == jnp.arange(8, dtype=species.dtype)[None, :])
    T = jnp.dot(onehot.astype(jnp.float32), species_embedding)
    T128 = jnp.concatenate([T, jnp.zeros((T.shape[0], 124), jnp.float32)], axis=1)
    spec128 = _sc_gather(T128, j)
    e0, e1, e2, e3 = _edge_features(R_ij, spec128)
    acc = _sc_scatter(e0, e1, e2, e3, i)

    N = species.shape[0]
    leaves = []
    for l in range(_MAX_ANGULAR + 1):
        m, n = _M_PER_L[l], _N_PER_L[l]
        off = int(_KOFF_L[l])
        sl = [a[:N, off:off + m * n] for a in acc]
        leaf = jnp.stack(sl, axis=-1).reshape(N, m, n, 4)
        leaves.append(leaf)
    return tuple(leaves)


# speedup vs baseline: 1.7493x; 1.2335x over previous
import functools

import jax
import jax.numpy as jnp
import numpy as np
from jax import lax
from jax.experimental import pallas as pl
from jax.experimental.pallas import tpu as pltpu
from jax.experimental.pallas import tpu_sc as plsc

_CUTOFF = 5.0
_MAX_RADIAL = 8
_MAX_ANGULAR = 3


def _np_jn(l, x):
    x = np.asarray(x, dtype=np.float64)
    j0 = np.sin(x) / x
    if l == 0:
        return j0
    j1 = np.sin(x) / x**2 - np.cos(x) / x
    if l == 1:
        return j1
    jm, jc = j0, j1
    for ll in range(1, l):
        jm, jc = jc, (2 * ll + 1) / x * jc - jm
    return jc


def _np_bessel_zeros(l, z_max):
    xs = np.arange(0.5, z_max + 1.0, 0.005)
    vals = _np_jn(l, xs)
    idx = np.where(np.sign(vals[:-1]) * np.sign(vals[1:]) < 0)[0]
    zeros = []
    for k in idx:
        a, b = xs[k], xs[k + 1]
        fa = _np_jn(l, a)
        for _ in range(60):
            m = 0.5 * (a + b)
            fm = _np_jn(l, m)
            if fa * fm <= 0:
                b = m
            else:
                a, fa = m, fm
        z = 0.5 * (a + b)
        if z <= z_max:
            zeros.append(z)
    return np.asarray(zeros, dtype=np.float64)


_ZMAX = _MAX_RADIAL * np.pi + 1e-6
_ZEROS = [_np_bessel_zeros(l, _ZMAX) for l in range(_MAX_ANGULAR + 1)]
_N_PER_L = [int(z.shape[0]) for z in _ZEROS]
_M_PER_L = [2 * l + 1 for l in range(_MAX_ANGULAR + 1)]
_NORMS = [
    np.sqrt(2.0 / _CUTOFF**3) / np.abs(_np_jn(l + 1, _ZEROS[l]))
    for l in range(_MAX_ANGULAR + 1)
]
_NRAD = sum(_N_PER_L)
_NANG = sum(_M_PER_L)
_K = sum(m * n for m, n in zip(_M_PER_L, _N_PER_L))

_Z_ROW = np.concatenate(_ZEROS).astype(np.float32)[None, :]
_NRM_ROW = np.concatenate(_NORMS).astype(np.float32)[None, :]
_L_ROW = np.concatenate(
    [np.full(n, l, np.float32) for l, n in enumerate(_N_PER_L)]
)[None, :]

_KOFF_L = np.cumsum([0] + [m * n for m, n in zip(_M_PER_L, _N_PER_L)])
_ROFF_L = np.cumsum([0] + _N_PER_L)
_AOFF_L = np.cumsum([0] + _M_PER_L)

_REP_R = np.zeros((_NRAD, _K), np.float32)
_REP_A = np.zeros((_NANG, _K), np.float32)
for _l in range(_MAX_ANGULAR + 1):
    _m, _n = _M_PER_L[_l], _N_PER_L[_l]
    for _mm in range(_m):
        for _nn in range(_n):
            _k = _KOFF_L[_l] + _mm * _n + _nn
            _REP_R[_ROFF_L[_l] + _nn, _k] = 1.0
            _REP_A[_AOFF_L[_l] + _mm, _k] = 1.0

_KP = 128

_PK = 4
_RW = _NRAD * _PK
_OW = _KP * _PK

_Z28 = np.concatenate(_ZEROS).astype(np.float64)
_L28 = np.concatenate([np.full(n, l) for l, n in enumerate(_N_PER_L)])
_NRM28 = np.concatenate(_NORMS).astype(np.float64)

_S12 = np.zeros((12, _PK), np.float32)
for _i in range(_PK):
    _S12[3 * _i:3 * _i + 3, _i] = 1.0

_CB = np.zeros((_PK, _RW), np.float32)
for _i in range(_PK):
    _CB[_i, 28 * _i:28 * _i + 28] = 1.0
_ZROW112 = np.tile((_Z28 / _CUTOFF).astype(np.float32), _PK)[None, :]

_WLR = np.zeros((4, _RW), np.float32)
for _L in range(4):
    _row = np.where(_L28 == _L, _NRM28, 0.0).astype(np.float32)
    for _i in range(_PK):
        _WLR[_L, 28 * _i:28 * _i + 28] = _row

_CT3 = np.zeros((3, _RW), np.float32)
_DT3 = np.zeros((3, _RW), np.float32)
for _t in range(3):
    _ge = (_L28 >= _t + 1).astype(np.float32)
    _CT3[_t, :] = np.tile(_ge, _PK)
    _DT3[_t, :] = np.tile(1.0 - _ge, _PK)

_REP_R4 = np.zeros((_RW, _OW), np.float32)
for _i in range(_PK):
    _REP_R4[28 * _i:28 * _i + 28, 128 * _i:128 * _i + _K] = _REP_R

def _poly_tables():
    def mono(e, c=1.0):
        return {e: c}

    def padd(a, b):
        out = dict(a)
        for k, v in b.items():
            out[k] = out.get(k, 0.0) + v
        return out

    def pmul(a, b):
        out = {}
        for ka, va in a.items():
            for kb, vb in b.items():
                k = tuple(x + y for x, y in zip(ka, kb))
                out[k] = out.get(k, 0.0) + va * vb
        return out

    def pscale(a, s):
        return {k: v * s for k, v in a.items()}

    X, Y, Z = mono((1, 0, 0)), mono((0, 1, 0)), mono((0, 0, 1))
    RR = padd(padd(pmul(X, X), pmul(Y, Y)), pmul(Z, Z))
    c1 = 0.4886025119029199
    c2a = 1.0925484305920792
    sh = [
        {(0, 0, 0): 0.28209479177387814},
        pscale(Y, c1), pscale(Z, c1), pscale(X, c1),
        pscale(pmul(X, Y), c2a),
        pscale(pmul(Y, Z), c2a),
        pscale(padd(pscale(pmul(Z, Z), 3.0), pscale(RR, -1.0)),
               0.31539156525252005),
        pscale(pmul(X, Z), c2a),
        pscale(padd(pmul(X, X), pscale(pmul(Y, Y), -1.0)), 0.5462742152960396),
        pscale(pmul(Y, padd(pscale(pmul(X, X), 3.0),
                            pscale(pmul(Y, Y), -1.0))), 0.5900435899266435),
        pscale(pmul(pmul(X, Y), Z), 2.890611442640554),
        pscale(pmul(Y, padd(pscale(pmul(Z, Z), 5.0), pscale(RR, -1.0))),
               0.4570457994644658),
        pscale(pmul(Z, padd(pscale(pmul(Z, Z), 5.0), pscale(RR, -3.0))),
               0.3731763325901154),
        pscale(pmul(X, padd(pscale(pmul(Z, Z), 5.0), pscale(RR, -1.0))),
               0.4570457994644658),
        pscale(pmul(Z, padd(pmul(X, X), pscale(pmul(Y, Y), -1.0))),
               1.445305721320277),
        pscale(pmul(X, padd(pmul(X, X), pscale(pmul(Y, Y), -3.0))),
               0.5900435899266435),
    ]
    return sh


_SH_POLY = _poly_tables()
_MONO3 = [(3, 0, 0), (2, 1, 0), (2, 0, 1), (1, 2, 0), (1, 1, 1),
          (1, 0, 2), (0, 3, 0), (0, 2, 1), (0, 1, 2), (0, 0, 3)]
_MONO2_IDX = {(2, 0, 0): 0, (1, 1, 0): 1, (1, 0, 1): 2,
              (0, 2, 0): 4, (0, 1, 1): 5, (0, 0, 2): 8}
_M3_SEL2 = [0, 0, 0, 4, 1, 8, 4, 4, 8, 8]
_M3_SEL1 = [0, 1, 2, 0, 2, 0, 1, 2, 1, 2]

_EA = np.zeros((12, 9 * _PK), np.float32)
_EB = np.zeros((12, 9 * _PK), np.float32)
for _i in range(_PK):
    for _a in range(3):
        for _b in range(3):
            _t = 3 * _a + _b
            _EA[3 * _i + _a, 9 * _i + _t] = 1.0
            _EB[3 * _i + _b, 9 * _i + _t] = 1.0

_FA = np.zeros((9 * _PK, 10 * _PK), np.float32)
_FB = np.zeros((12, 10 * _PK), np.float32)
for _i in range(_PK):
    for _t in range(10):
        _FA[9 * _i + _M3_SEL2[_t], 10 * _i + _t] = 1.0
        _FB[3 * _i + _M3_SEL1[_t], 10 * _i + _t] = 1.0

_ANG_OF_K = np.argmax(_REP_A, axis=0)
_C0ROW = np.zeros((1, _OW), np.float32)
_W1B = np.zeros((12, _OW), np.float32)
_W2B = np.zeros((9 * _PK, _OW), np.float32)
_W3B = np.zeros((10 * _PK, _OW), np.float32)
for _i in range(_PK):
    for _k in range(_K):
        _p = _SH_POLY[int(_ANG_OF_K[_k])]
        _col = 128 * _i + _k
        for _e, _cf in _p.items():
            _d = sum(_e)
            if _d == 0:
                _C0ROW[0, _col] += _cf
            elif _d == 1:
                _W1B[3 * _i + _e.index(1), _col] += _cf
            elif _d == 2:
                _W2B[9 * _i + _MONO2_IDX[_e], _col] += _cf
            else:
                _t3 = _MONO3.index(_e)
                _W3B[10 * _i + _t3, _col] += _cf

_G512 = np.zeros((4, _OW), np.float32)
for _c in range(4):
    _G512[_c, 128 * _c:128 * _c + 128] = 1.0


def _bf_split_np(m):
    hi = m.astype(np.float32).astype(jnp.bfloat16).astype(np.float32)
    return hi, (m - hi).astype(np.float32)


_W1B_H, _W1B_L = _bf_split_np(_W1B)
_W2B_H, _W2B_L = _bf_split_np(_W2B)
_W3B_H, _W3B_L = _bf_split_np(_W3B)

_P64 = np.float64(np.pi / 2)
_SC_P1 = np.float32(np.round(_P64 * 4096) / 4096)
_SC_P2 = np.float32(np.round((_P64 - np.float64(_SC_P1)) * 4096 ** 3)
                    / 4096 ** 3)
_SC_P3 = np.float32(_P64 - np.float64(_SC_P1) - np.float64(_SC_P2))
_TWO_OVER_PI = np.float32(2.0 / np.pi)

_EDGE_BLOCK = 1280
_N_EDGES = 160000
_N_NODES = 10000
_ACC_ROWS = 10240
_STRIPE = _ACC_ROWS // 16
_EPT = _N_EDGES // 16
_CHUNK = 128
_NFULL = _EPT // _CHUNK
_TAIL = _EPT - _NFULL * _CHUNK


def _dot(a, b):
    return jnp.dot(a, b, preferred_element_type=jnp.float32)


def _bsplit(a):
    hi = a.astype(jnp.bfloat16).astype(jnp.float32)
    return hi, a - hi


def _pick(ah, al, m):
    return _dot(ah, m) + _dot(al, m)


def _coef(ah, al, mh, ml):
    return _dot(ah, mh) + _dot(al, mh) + _dot(ah, ml)


def _sincos(xr):
    k = jnp.round(xr * _TWO_OVER_PI)
    ki = k.astype(jnp.int32)
    y = ((xr - k * _SC_P1) - k * _SC_P2) - k * _SC_P3
    y2 = y * y
    sp = y * (1.0 + y2 * (np.float32(-1 / 6) + y2 * (np.float32(1 / 120)
              + y2 * (np.float32(-1 / 5040) + y2 * np.float32(1 / 362880)))))
    cp = 1.0 + y2 * (np.float32(-0.5) + y2 * (np.float32(1 / 24)
              + y2 * (np.float32(-1 / 720) + y2 * np.float32(1 / 40320))))
    swap = (ki & 1) == 1
    s_sgn = jnp.where((ki & 2) == 2, -1.0, 1.0)
    c_sgn = jnp.where(((ki + 1) & 2) == 2, -1.0, 1.0)
    return s_sgn * jnp.where(swap, cp, sp), c_sgn * jnp.where(swap, sp, cp)


def _edge_kernel(r12_ref, spec_ref, s12_ref, cb_ref, zrow_ref, wlr_ref,
                 ct3_ref, dt3_ref, repr4_ref, ea_ref, eb_ref, fa_ref, fb_ref,
                 c0_ref, w1bh_ref, w1bl_ref, w2bh_ref, w2bl_ref, w3bh_ref,
                 w3bl_ref, g512_ref, e0_ref, e1_ref, e2_ref, e3_ref):
    B = e0_ref.shape[0]
    R12 = r12_ref[...]
    Rh, Rl = _bsplit(R12)
    sqh, sql = _bsplit(R12 * R12)
    r2 = _pick(sqh, sql, s12_ref[...])
    r = jnp.sqrt(r2)
    rinv = 1.0 / jnp.maximum(r, 1e-9)
    cb = cb_ref[...]

    rh, rl = _bsplit(r)
    r_ext = _pick(rh, rl, cb)
    xr = jnp.maximum(r_ext * zrow_ref[...], 1e-2)
    s, c = _sincos(xr)
    inv = 1.0 / xr
    j0 = s * inv
    j1 = s * inv * inv - c * inv
    j2 = 3.0 * inv * j1 - j0
    j3 = 5.0 * inv * j2 - j1
    wlr = wlr_ref[...]
    rad = (j0 * wlr[0:1, :] + j1 * wlr[1:2, :]
           + j2 * wlr[2:3, :] + j3 * wlr[3:4, :])
    rih, ril = _bsplit(rinv)
    ri_ext = _pick(rih, ril, cb)
    ct3 = ct3_ref[...]
    dt3 = dt3_ref[...]
    f = ((ri_ext * ct3[0:1, :] + dt3[0:1, :])
         * (ri_ext * ct3[1:2, :] + dt3[1:2, :])
         * (ri_ext * ct3[2:3, :] + dt3[2:3, :]))
    radh, radl_ = _bsplit(rad * f)
    rad_ext = _pick(radh, radl_, repr4_ref[...])

    m2 = _pick(Rh, Rl, ea_ref[...]) * _pick(Rh, Rl, eb_ref[...])
    m2h, m2l = _bsplit(m2)
    m3 = _pick(m2h, m2l, fa_ref[...]) * _pick(Rh, Rl, fb_ref[...])
    m3h, m3l = _bsplit(m3)
    ang = (c0_ref[...]
           + _coef(Rh, Rl, w1bh_ref[...], w1bl_ref[...])
           + _coef(m2h, m2l, w2bh_ref[...], w2bl_ref[...])
           + _coef(m3h, m3l, w3bh_ref[...], w3bl_ref[...]))

    u = jnp.reshape(rad_ext * ang, (B, _KP))
    spec4 = spec_ref[...][:, 0:4]
    sph, spl = _bsplit(spec4)
    sb = _pick(sph, spl, g512_ref[...])
    e0_ref[...] = u * sb[:, 0:128]
    e1_ref[...] = u * sb[:, 128:256]
    e2_ref[...] = u * sb[:, 256:384]
    e3_ref[...] = u * sb[:, 384:512]


def _const_spec(shape):
    return pl.BlockSpec(shape, lambda b: (0,) * len(shape))


def _edge_features(R_ij, spec_e):
    P = R_ij.shape[0]
    B = _EDGE_BLOCK
    B4 = B // _PK
    grid = (P // B,)
    R12 = R_ij.reshape(P // _PK, 3 * _PK)
    out_spec = pl.BlockSpec((B, _KP), lambda b: (b, 0))
    e_shape = jax.ShapeDtypeStruct((P, _KP), jnp.float32)

    return pl.pallas_call(
        _edge_kernel,
        grid=grid,
        in_specs=[
            pl.BlockSpec((B4, 12), lambda b: (b, 0)),
            pl.BlockSpec((B, 128), lambda b: (b, 0)),
            _const_spec((12, _PK)),
            _const_spec((_PK, _RW)),
            _const_spec((1, _RW)),
            _const_spec((4, _RW)),
            _const_spec((3, _RW)),
            _const_spec((3, _RW)),
            _const_spec((_RW, _OW)),
            _const_spec((12, 9 * _PK)),
            _const_spec((12, 9 * _PK)),
            _const_spec((9 * _PK, 10 * _PK)),
            _const_spec((12, 10 * _PK)),
            _const_spec((1, _OW)),
            _const_spec((12, _OW)),
            _const_spec((12, _OW)),
            _const_spec((9 * _PK, _OW)),
            _const_spec((9 * _PK, _OW)),
            _const_spec((10 * _PK, _OW)),
            _const_spec((10 * _PK, _OW)),
            _const_spec((4, _OW)),
        ],
        out_specs=[out_spec] * 4,
        out_shape=[e_shape] * 4,
    )(
        R12,
        spec_e,
        jnp.asarray(_S12),
        jnp.asarray(_CB),
        jnp.asarray(_ZROW112),
        jnp.asarray(_WLR),
        jnp.asarray(_CT3),
        jnp.asarray(_DT3),
        jnp.asarray(_REP_R4),
        jnp.asarray(_EA),
        jnp.asarray(_EB),
        jnp.asarray(_FA),
        jnp.asarray(_FB),
        jnp.asarray(_C0ROW),
        jnp.asarray(_W1B_H),
        jnp.asarray(_W1B_L),
        jnp.asarray(_W2B_H),
        jnp.asarray(_W2B_L),
        jnp.asarray(_W3B_H),
        jnp.asarray(_W3B_L),
        jnp.asarray(_G512),
    )


_GW = 32
_GEPT = _N_EDGES // _GW
_GCHUNK = 128
_GNFULL = _GEPT // _GCHUNK
_GTAIL = _GEPT - _GNFULL * _GCHUNK


def _sc_gather_body(tbl_hbm, idx_hbm, out_hbm, idx_v, rows_v, idx_t, rows_t,
                    sem):
    cid = lax.axis_index("c")
    sid = lax.axis_index("s")
    wid = sid * 2 + cid
    base = wid * _GEPT

    def body(g, _):
        off = base + g * _GCHUNK
        pltpu.sync_copy(idx_hbm.at[pl.ds(off, _GCHUNK)], idx_v)
        pltpu.async_copy(tbl_hbm.at[idx_v], rows_v, sem).wait()
        pltpu.sync_copy(rows_v, out_hbm.at[pl.ds(off, _GCHUNK)])
        return _

    lax.fori_loop(0, _GNFULL, body, 0)
    toff = base + _GNFULL * _GCHUNK
    pltpu.sync_copy(idx_hbm.at[pl.ds(toff, _GTAIL)], idx_t)
    pltpu.async_copy(tbl_hbm.at[idx_t], rows_t, sem).wait()
    pltpu.sync_copy(rows_t, out_hbm.at[pl.ds(toff, _GTAIL)])


def _sc_gather(T16, j):
    mesh = plsc.VectorSubcoreMesh(core_axis_name="c", subcore_axis_name="s")
    f = pl.kernel(
        _sc_gather_body,
        mesh=mesh,
        out_type=jax.ShapeDtypeStruct((_N_EDGES, 128), jnp.float32),
        scratch_types=[
            pltpu.VMEM((_GCHUNK,), jnp.int32),
            pltpu.VMEM((_GCHUNK, 128), jnp.float32),
            pltpu.VMEM((_GTAIL,), jnp.int32),
            pltpu.VMEM((_GTAIL, 128), jnp.float32),
            pltpu.SemaphoreType.DMA,
        ],
    )
    return f(T16, j)


def _sc_scatter_body(e0, e1, e2, e3, idx_hbm, zeros_hbm,
                     out0, out1, out2, out3,
                     idx_v, rows_v, idx_v2, rows_v2, idx_t, rows_t, acc_sh,
                     sem_i0, sem_r0, sem_i1, sem_r1):
    cid = lax.axis_index("c")
    sid = lax.axis_index("s")
    e_refs = (e0, e1, e2, e3)
    out_refs = (out0, out1, out2, out3)
    base_t = sid * _EPT

    for pass_k in range(2):
        pltpu.sync_copy(zeros_hbm, acc_sh.at[pl.ds(sid * _STRIPE, _STRIPE)])
        plsc.subcore_barrier()

        for which in range(2):
            ch = 2 * pass_k + which
            e_ref = e_refs[ch]

            @pl.when(cid == which)
            def _scatter(e_ref=e_ref):
                bufs = ((idx_v, rows_v, sem_i0, sem_r0),
                        (idx_v2, rows_v2, sem_i1, sem_r1))

                def start(g, b):
                    off = base_t + g * _CHUNK
                    pltpu.async_copy(idx_hbm.at[pl.ds(off, _CHUNK)], b[0], b[2])
                    pltpu.async_copy(e_ref.at[pl.ds(off, _CHUNK)], b[1], b[3])

                def finish(g, b):
                    off = base_t + g * _CHUNK
                    pltpu.make_async_copy(
                        idx_hbm.at[pl.ds(off, _CHUNK)], b[0], b[2]).wait()
                    pltpu.make_async_copy(
                        e_ref.at[pl.ds(off, _CHUNK)], b[1], b[3]).wait()

                start(0, bufs[0])
                start(1, bufs[1])

                def body(g2, _):
                    g = 2 * g2
                    for half in range(2):
                        b = bufs[half]
                        finish(g + half, b)
                        pltpu.sync_copy(b[1], acc_sh.at[b[0]], add=True)

                        @pl.when(g2 < _NFULL // 2 - 1)
                        def _next(b=b, g=g, half=half):
                            start(g + 2 + half, b)

                    return _

                lax.fori_loop(0, _NFULL // 2, body, 0)
                toff = base_t + _NFULL * _CHUNK
                pltpu.sync_copy(idx_hbm.at[pl.ds(toff, _TAIL)], idx_t)
                pltpu.sync_copy(e_ref.at[pl.ds(toff, _TAIL)], rows_t)
                pltpu.sync_copy(rows_t, acc_sh.at[idx_t], add=True)

        plsc.subcore_barrier()

        for which in range(2):
            ch = 2 * pass_k + which
            out_ref = out_refs[ch]

            @pl.when(cid == which)
            def _flush(out_ref=out_ref):
                s = pl.ds(sid * _STRIPE, _STRIPE)
                pltpu.sync_copy(acc_sh.at[s], out_ref.at[s])

        plsc.subcore_barrier()


def _sc_scatter(e0, e1, e2, e3, i):
    zeros = jnp.zeros((_STRIPE, _KP), jnp.float32)
    acc_ty = jax.ShapeDtypeStruct((_ACC_ROWS, _KP), jnp.float32)
    mesh = plsc.VectorSubcoreMesh(core_axis_name="c", subcore_axis_name="s")
    f = pl.kernel(
        _sc_scatter_body,
        mesh=mesh,
        out_type=[acc_ty] * 4,
        scratch_types=[
            pltpu.VMEM((_CHUNK,), jnp.int32),
            pltpu.VMEM((_CHUNK, _KP), jnp.float32),
            pltpu.VMEM((_CHUNK,), jnp.int32),
            pltpu.VMEM((_CHUNK, _KP), jnp.float32),
            pltpu.VMEM((_TAIL,), jnp.int32),
            pltpu.VMEM((_TAIL, _KP), jnp.float32),
            pltpu.VMEM_SHARED((_ACC_ROWS, _KP), jnp.float32),
            pltpu.SemaphoreType.DMA,
            pltpu.SemaphoreType.DMA,
            pltpu.SemaphoreType.DMA,
            pltpu.SemaphoreType.DMA,
        ],
    )
    return f(e0, e1, e2, e3, i, zeros)


@functools.partial(jax.jit)
def kernel(R_ij, i, j, species, species_embedding):
    onehot = (species[:, None] == jnp.arange(8, dtype=species.dtype)[None, :])
    T = jnp.dot(onehot.astype(jnp.float32), species_embedding)
    T128 = jnp.concatenate([T, jnp.zeros((T.shape[0], 124), jnp.float32)], axis=1)
    spec128 = _sc_gather(T128, j)
    e0, e1, e2, e3 = _edge_features(R_ij, spec128)
    acc = _sc_scatter(e0, e1, e2, e3, i)

    N = species.shape[0]
    leaves = []
    for l in range(_MAX_ANGULAR + 1):
        m, n = _M_PER_L[l], _N_PER_L[l]
        off = int(_KOFF_L[l])
        sl = [a[:N, off:off + m * n] for a in acc]
        leaf = jnp.stack(sl, axis=-1).reshape(N, m, n, 4)
        leaves.append(leaf)
    return tuple(leaves)
